# baseline (speedup 1.0000x reference)
import functools

import jax
import jax.numpy as jnp
from jax import lax
from jax.experimental import pallas as pl
from jax.experimental.pallas import tpu as pltpu
from jax.experimental.pallas import tpu_sc as plsc

_NC, _NS, _L = 2, 16, 16
_NW = _NC * _NS
_MOM = 0.5


def _make_sc_kernel(B, KP1, D, N):
    b_per_w = B // _NW
    half = KP1 // 2
    n_g = half // _L
    n_qc = D // _L
    mesh = plsc.VectorSubcoreMesh(core_axis_name="c", subcore_axis_name="s",
                                  num_cores=_NC, num_subcores=_NS)

    @functools.partial(
        pl.kernel,
        out_type=(
            jax.ShapeDtypeStruct((B, KP1), jnp.float32),
            jax.ShapeDtypeStruct((B, KP1), jnp.float32),
            jax.ShapeDtypeStruct((B, D), jnp.float32),
            jax.ShapeDtypeStruct((B, D), jnp.float32),
        ),
        mesh=mesh,
        compiler_params=pltpu.CompilerParams(needs_layout_passes=False),
        scratch_types=[
            pltpu.VMEM((b_per_w, KP1 // 128, 128), jnp.int32),
            pltpu.VMEM((KP1 // 128, 128), jnp.int32),
            pltpu.VMEM((b_per_w, D), jnp.float32),
            pltpu.VMEM((b_per_w, D), jnp.float32),
            pltpu.VMEM((half, D), jnp.float32),
            pltpu.VMEM((half, D), jnp.float32),
            pltpu.VMEM((_L * _L,), jnp.float32),
            pltpu.VMEM((KP1,), jnp.float32),
            pltpu.VMEM((b_per_w,), jnp.int32),
            pltpu.VMEM((b_per_w, D), jnp.float32),
            pltpu.SemaphoreType.DMA,
        ],
    )
    def sc_kernel(mem_gene_hbm, mem_image_hbm, image_hbm, gene_hbm, idx_hbm,
                  index_hbm, dots_i_hbm, dots_g_hbm, old_i_hbm, old_g_hbm,
                  idx_all, idx_cur, qi_all, qg_all, buf0, buf1, tr_v, dots_v,
                  iidx_v, orow_v, sem):
        cid = lax.axis_index("c")
        sid = lax.axis_index("s")
        wid = sid * _NC + cid
        iota = lax.iota(jnp.int32, _L)
        tr_idx = iota * _L

        pltpu.sync_copy(idx_hbm.at[wid], idx_all)
        pltpu.sync_copy(image_hbm.at[pl.ds(wid * b_per_w, b_per_w)], qi_all)
        pltpu.sync_copy(gene_hbm.at[pl.ds(wid * b_per_w, b_per_w)], qg_all)

        def half_descs(bank_hbm, h, buf):
            return [
                pltpu.make_async_copy(bank_hbm.at[idx_cur.at[2 * h + jj]],
                                      buf.at[pl.ds(jj * 128, 128)], sem)
                for jj in range(2)
            ]

        def compute(buf, qc, h):
            def g_body(g, c2):
                for j in range(_L):
                    k = g * _L + j
                    acc = buf[k, pl.ds(0, _L)] * qc[0]
                    for c in range(1, n_qc):
                        acc = acc + buf[k, pl.ds(c * _L, _L)] * qc[c]
                    plsc.store_scatter(tr_v, [tr_idx + j], acc)
                vec = tr_v[pl.ds(0, _L)]
                for m in range(1, _L):
                    vec = vec + tr_v[pl.ds(m * _L, _L)]
                dots_v[pl.ds(h * half + g * _L, _L)] = vec
                return c2

            lax.fori_loop(0, n_g, g_body, 0)

        def stage_idx(i):
            for j in range(KP1 // 128):
                for l in range(128 // _L):
                    idx_cur[j, pl.ds(l * _L, _L)] = idx_all[i, j,
                                                            pl.ds(l * _L, _L)]

        def do_bank(bank_hbm, q_all, dots_hbm):
            stage_idx(0)
            for dsc in half_descs(bank_hbm, 0, buf0):
                dsc.start()

            def pair(i, carry):
                qc = [q_all[i, pl.ds(c * _L, _L)] for c in range(n_qc)]
                for dsc in half_descs(bank_hbm, 0, buf0):
                    dsc.wait()
                for dsc in half_descs(bank_hbm, 1, buf1):
                    dsc.start()
                compute(buf0, qc, 0)
                for dsc in half_descs(bank_hbm, 1, buf1):
                    dsc.wait()

                @pl.when(i + 1 < b_per_w)
                def _prefetch():
                    stage_idx(i + 1)
                    for dsc in half_descs(bank_hbm, 0, buf0):
                        dsc.start()

                compute(buf1, qc, 1)
                pltpu.sync_copy(dots_v, dots_hbm.at[wid * b_per_w + i])
                return carry

            lax.fori_loop(0, b_per_w, pair, 0)

        do_bank(mem_gene_hbm, qi_all, dots_i_hbm)
        do_bank(mem_image_hbm, qg_all, dots_g_hbm)

        pltpu.sync_copy(index_hbm.at[wid], iidx_v)
        pltpu.async_copy(mem_image_hbm.at[iidx_v], orow_v, sem).wait()
        pltpu.sync_copy(orow_v, old_i_hbm.at[pl.ds(wid * b_per_w, b_per_w)])
        pltpu.async_copy(mem_gene_hbm.at[iidx_v], orow_v, sem).wait()
        pltpu.sync_copy(orow_v, old_g_hbm.at[pl.ds(wid * b_per_w, b_per_w)])

    return sc_kernel


def _make_bank_copy(N, D):
    n_chunks = 32
    rows = N // n_chunks

    def body(src_i_ref, src_g_ref, dst_i_ref, dst_g_ref, sem_i, sem_g):
        def descs(k):
            s = pl.ds(k * rows, rows)
            return (pltpu.make_async_copy(src_i_ref.at[s], dst_i_ref.at[s],
                                          sem_i),
                    pltpu.make_async_copy(src_g_ref.at[s], dst_g_ref.at[s],
                                          sem_g))

        def fire(k, c):
            for d in descs(k):
                d.start()
            return c

        lax.fori_loop(0, n_chunks, fire, 0)

        def drain(k, c):
            for d in descs(k):
                d.wait()
            return c

        lax.fori_loop(0, n_chunks, drain, 0)

    return pl.pallas_call(
        body,
        out_shape=(
            jax.ShapeDtypeStruct((N, D), jnp.float32),
            jax.ShapeDtypeStruct((N, D), jnp.float32),
        ),
        in_specs=[pl.BlockSpec(memory_space=pl.ANY)] * 2,
        out_specs=(pl.BlockSpec(memory_space=pl.ANY),) * 2,
        scratch_shapes=[pltpu.SemaphoreType.DMA, pltpu.SemaphoreType.DMA],
    )


def _make_epilogue(B, KP1, D, N):
    def body(dots_i_ref, dots_g_ref, old_i_ref, old_g_ref, img_ref, gene_ref,
             index_ref, bank_i_ref, bank_g_ref,
             out_i_ref, out_g_ref, newb_i_ref, newb_g_ref,
             upd_i_scr, upd_g_scr, sem_i, sem_g):
        del bank_i_ref, bank_g_ref
        for dref, oref in ((dots_i_ref, out_i_ref), (dots_g_ref, out_g_ref)):
            x = dref[...]
            m = jnp.max(x, axis=1, keepdims=True)
            e = jnp.exp(x - m)
            oref[...] = e / jnp.sum(e, axis=1, keepdims=True)
        for old_ref, v_ref, upd_scr in ((old_i_ref, img_ref, upd_i_scr),
                                        (old_g_ref, gene_ref, upd_g_scr)):
            pos = old_ref[...] * _MOM + v_ref[...] * (1.0 - _MOM)
            nrm = jnp.sqrt(jnp.sum(pos * pos, axis=1, keepdims=True))
            upd_scr[...] = pos / nrm

        def fire(i, c):
            tgt = index_ref[i]
            pltpu.make_async_copy(upd_i_scr.at[i], newb_i_ref.at[tgt], sem_i).start()
            pltpu.make_async_copy(upd_g_scr.at[i], newb_g_ref.at[tgt], sem_g).start()
            return c

        lax.fori_loop(0, B, fire, 0)

        def drain(i, c):
            tgt = index_ref[i]
            pltpu.make_async_copy(upd_i_scr.at[i], newb_i_ref.at[tgt], sem_i).wait()
            pltpu.make_async_copy(upd_g_scr.at[i], newb_g_ref.at[tgt], sem_g).wait()
            return c

        lax.fori_loop(0, B, drain, 0)

    return pl.pallas_call(
        body,
        out_shape=(
            jax.ShapeDtypeStruct((B, KP1), jnp.float32),
            jax.ShapeDtypeStruct((B, KP1), jnp.float32),
            jax.ShapeDtypeStruct((N, D), jnp.float32),
            jax.ShapeDtypeStruct((N, D), jnp.float32),
        ),
        in_specs=[
            pl.BlockSpec(memory_space=pltpu.VMEM),
            pl.BlockSpec(memory_space=pltpu.VMEM),
            pl.BlockSpec(memory_space=pltpu.VMEM),
            pl.BlockSpec(memory_space=pltpu.VMEM),
            pl.BlockSpec(memory_space=pltpu.VMEM),
            pl.BlockSpec(memory_space=pltpu.VMEM),
            pl.BlockSpec(memory_space=pltpu.SMEM),
            pl.BlockSpec(memory_space=pl.ANY),
            pl.BlockSpec(memory_space=pl.ANY),
        ],
        out_specs=(
            pl.BlockSpec(memory_space=pltpu.VMEM),
            pl.BlockSpec(memory_space=pltpu.VMEM),
            pl.BlockSpec(memory_space=pl.ANY),
            pl.BlockSpec(memory_space=pl.ANY),
        ),
        scratch_shapes=[
            pltpu.VMEM((B, D), jnp.float32),
            pltpu.VMEM((B, D), jnp.float32),
            pltpu.SemaphoreType.DMA,
            pltpu.SemaphoreType.DMA,
        ],
        input_output_aliases={7: 2, 8: 3},
    )


def kernel(image, gene, index, idx, memory_image, memory_gene):
    B, D = image.shape
    KP1 = idx.shape[1]
    N = memory_image.shape[0]
    idx3 = idx.reshape(_NW, B // _NW, KP1 // 128, 128)
    index2 = index.reshape(_NW, B // _NW)

    sc = _make_sc_kernel(B, KP1, D, N)
    dots_i, dots_g, old_i, old_g = sc(memory_gene, memory_image, image, gene,
                                      idx3, index2)
    cpy = _make_bank_copy(N, D)
    cpy_i, cpy_g = cpy(memory_image, memory_gene)
    epi = _make_epilogue(B, KP1, D, N)
    out_i, out_g, new_i, new_g = epi(dots_i, dots_g, old_i, old_g, image, gene,
                                     index, cpy_i, cpy_g)
    return (out_i[:, :, None], out_g[:, :, None], new_i, new_g)


# speedup vs baseline: 36.1782x; 36.1782x over previous
import functools

import jax
import jax.numpy as jnp
from jax import lax
from jax.experimental import pallas as pl
from jax.experimental.pallas import tpu as pltpu
from jax.experimental.pallas import tpu_sc as plsc

_NC, _NS, _L = 2, 16, 16
_NW = _NC * _NS
_MOM = 0.5


def _make_sc_kernel(B, KP1, D, N):
    b_per_w = B // _NW
    half = KP1 // 2
    n_g = half // _L
    n_qc = D // _L
    mesh = plsc.VectorSubcoreMesh(core_axis_name="c", subcore_axis_name="s",
                                  num_cores=_NC, num_subcores=_NS)

    @functools.partial(
        pl.kernel,
        out_type=(
            jax.ShapeDtypeStruct((B, KP1), jnp.float32),
            jax.ShapeDtypeStruct((B, KP1), jnp.float32),
            jax.ShapeDtypeStruct((B, D), jnp.float32),
            jax.ShapeDtypeStruct((B, D), jnp.float32),
        ),
        mesh=mesh,
        compiler_params=pltpu.CompilerParams(needs_layout_passes=False),
        scratch_types=[
            pltpu.VMEM((b_per_w, KP1 // 128, 128), jnp.int32),
            pltpu.VMEM((KP1 // 128, 128), jnp.int32),
            pltpu.VMEM((b_per_w, D), jnp.float32),
            pltpu.VMEM((b_per_w, D), jnp.float32),
            pltpu.VMEM((half, D), jnp.float32),
            pltpu.VMEM((half, D), jnp.float32),
            pltpu.VMEM((_L * _L,), jnp.float32),
            pltpu.VMEM((KP1,), jnp.float32),
            pltpu.VMEM((b_per_w,), jnp.int32),
            pltpu.VMEM((b_per_w, D), jnp.float32),
            pltpu.SemaphoreType.DMA,
        ],
    )
    def sc_kernel(mem_gene_hbm, mem_image_hbm, image_hbm, gene_hbm, idx_hbm,
                  index_hbm, dots_i_hbm, dots_g_hbm, old_i_hbm, old_g_hbm,
                  idx_all, idx_cur, qi_all, qg_all, buf0, buf1, tr_v, dots_v,
                  iidx_v, orow_v, sem):
        cid = lax.axis_index("c")
        sid = lax.axis_index("s")
        wid = sid * _NC + cid
        iota = lax.iota(jnp.int32, _L)
        tr_idx = iota * _L

        pltpu.sync_copy(idx_hbm.at[wid], idx_all)
        pltpu.sync_copy(image_hbm.at[pl.ds(wid * b_per_w, b_per_w)], qi_all)
        pltpu.sync_copy(gene_hbm.at[pl.ds(wid * b_per_w, b_per_w)], qg_all)

        def half_descs(bank_hbm, h, buf):
            return [
                pltpu.make_async_copy(bank_hbm.at[idx_cur.at[2 * h + jj]],
                                      buf.at[pl.ds(jj * 128, 128)], sem)
                for jj in range(2)
            ]

        def compute(buf, qc, h):
            def g_body(g, c2):
                for j in range(_L):
                    k = g * _L + j
                    acc = buf[k, pl.ds(0, _L)] * qc[0]
                    for c in range(1, n_qc):
                        acc = acc + buf[k, pl.ds(c * _L, _L)] * qc[c]
                    plsc.store_scatter(tr_v, [tr_idx + j], acc)
                vec = tr_v[pl.ds(0, _L)]
                for m in range(1, _L):
                    vec = vec + tr_v[pl.ds(m * _L, _L)]
                dots_v[pl.ds(h * half + g * _L, _L)] = vec
                return c2

            lax.fori_loop(0, n_g, g_body, 0)

        def stage_idx(i):
            for j in range(KP1 // 128):
                for l in range(128 // _L):
                    idx_cur[j, pl.ds(l * _L, _L)] = idx_all[i, j,
                                                            pl.ds(l * _L, _L)]

        def do_bank(bank_hbm, q_all, dots_hbm):
            stage_idx(0)
            for dsc in half_descs(bank_hbm, 0, buf0):
                dsc.start()

            def pair(i, carry):
                qc = [q_all[i, pl.ds(c * _L, _L)] for c in range(n_qc)]
                for dsc in half_descs(bank_hbm, 0, buf0):
                    dsc.wait()
                for dsc in half_descs(bank_hbm, 1, buf1):
                    dsc.start()
                compute(buf0, qc, 0)
                for dsc in half_descs(bank_hbm, 1, buf1):
                    dsc.wait()

                @pl.when(i + 1 < b_per_w)
                def _prefetch():
                    stage_idx(i + 1)
                    for dsc in half_descs(bank_hbm, 0, buf0):
                        dsc.start()

                compute(buf1, qc, 1)
                pltpu.sync_copy(dots_v, dots_hbm.at[wid * b_per_w + i])
                return carry

            lax.fori_loop(0, b_per_w, pair, 0)

        do_bank(mem_gene_hbm, qi_all, dots_i_hbm)
        do_bank(mem_image_hbm, qg_all, dots_g_hbm)

        pltpu.sync_copy(index_hbm.at[wid], iidx_v)
        pltpu.async_copy(mem_image_hbm.at[iidx_v], orow_v, sem).wait()
        pltpu.sync_copy(orow_v, old_i_hbm.at[pl.ds(wid * b_per_w, b_per_w)])
        pltpu.async_copy(mem_gene_hbm.at[iidx_v], orow_v, sem).wait()
        pltpu.sync_copy(orow_v, old_g_hbm.at[pl.ds(wid * b_per_w, b_per_w)])

    return sc_kernel


def _make_bank_copy(N, D):
    rows = 8192
    grid = N // rows

    def body(src_i_ref, src_g_ref, dst_i_ref, dst_g_ref):
        dst_i_ref[...] = src_i_ref[...]
        dst_g_ref[...] = src_g_ref[...]

    spec = pl.BlockSpec((rows, D), lambda k: (k, 0))
    return pl.pallas_call(
        body,
        grid=(grid,),
        out_shape=(
            jax.ShapeDtypeStruct((N, D), jnp.float32),
            jax.ShapeDtypeStruct((N, D), jnp.float32),
        ),
        in_specs=[spec, spec],
        out_specs=(spec, spec),
        compiler_params=pltpu.CompilerParams(
            dimension_semantics=("arbitrary",)),
    )


def _make_epilogue(B, KP1, D, N):
    def body(dots_i_ref, dots_g_ref, old_i_ref, old_g_ref, img_ref, gene_ref,
             index_ref, bank_i_ref, bank_g_ref,
             out_i_ref, out_g_ref, newb_i_ref, newb_g_ref,
             upd_i_scr, upd_g_scr, sem_i, sem_g):
        del bank_i_ref, bank_g_ref
        for dref, oref in ((dots_i_ref, out_i_ref), (dots_g_ref, out_g_ref)):
            x = dref[...]
            m = jnp.max(x, axis=1, keepdims=True)
            e = jnp.exp(x - m)
            oref[...] = e / jnp.sum(e, axis=1, keepdims=True)
        for old_ref, v_ref, upd_scr in ((old_i_ref, img_ref, upd_i_scr),
                                        (old_g_ref, gene_ref, upd_g_scr)):
            pos = old_ref[...] * _MOM + v_ref[...] * (1.0 - _MOM)
            nrm = jnp.sqrt(jnp.sum(pos * pos, axis=1, keepdims=True))
            upd_scr[...] = pos / nrm

        def fire(i, c):
            tgt = index_ref[i]
            pltpu.make_async_copy(upd_i_scr.at[i], newb_i_ref.at[tgt], sem_i).start()
            pltpu.make_async_copy(upd_g_scr.at[i], newb_g_ref.at[tgt], sem_g).start()
            return c

        lax.fori_loop(0, B, fire, 0)

        def drain(i, c):
            tgt = index_ref[i]
            pltpu.make_async_copy(upd_i_scr.at[i], newb_i_ref.at[tgt], sem_i).wait()
            pltpu.make_async_copy(upd_g_scr.at[i], newb_g_ref.at[tgt], sem_g).wait()
            return c

        lax.fori_loop(0, B, drain, 0)

    return pl.pallas_call(
        body,
        out_shape=(
            jax.ShapeDtypeStruct((B, KP1), jnp.float32),
            jax.ShapeDtypeStruct((B, KP1), jnp.float32),
            jax.ShapeDtypeStruct((N, D), jnp.float32),
            jax.ShapeDtypeStruct((N, D), jnp.float32),
        ),
        in_specs=[
            pl.BlockSpec(memory_space=pltpu.VMEM),
            pl.BlockSpec(memory_space=pltpu.VMEM),
            pl.BlockSpec(memory_space=pltpu.VMEM),
            pl.BlockSpec(memory_space=pltpu.VMEM),
            pl.BlockSpec(memory_space=pltpu.VMEM),
            pl.BlockSpec(memory_space=pltpu.VMEM),
            pl.BlockSpec(memory_space=pltpu.SMEM),
            pl.BlockSpec(memory_space=pl.ANY),
            pl.BlockSpec(memory_space=pl.ANY),
        ],
        out_specs=(
            pl.BlockSpec(memory_space=pltpu.VMEM),
            pl.BlockSpec(memory_space=pltpu.VMEM),
            pl.BlockSpec(memory_space=pl.ANY),
            pl.BlockSpec(memory_space=pl.ANY),
        ),
        scratch_shapes=[
            pltpu.VMEM((B, D), jnp.float32),
            pltpu.VMEM((B, D), jnp.float32),
            pltpu.SemaphoreType.DMA,
            pltpu.SemaphoreType.DMA,
        ],
        input_output_aliases={7: 2, 8: 3},
    )


def kernel(image, gene, index, idx, memory_image, memory_gene):
    B, D = image.shape
    KP1 = idx.shape[1]
    N = memory_image.shape[0]
    idx3 = idx.reshape(_NW, B // _NW, KP1 // 128, 128)
    index2 = index.reshape(_NW, B // _NW)

    sc = _make_sc_kernel(B, KP1, D, N)
    dots_i, dots_g, old_i, old_g = sc(memory_gene, memory_image, image, gene,
                                      idx3, index2)
    cpy = _make_bank_copy(N, D)
    cpy_i, cpy_g = cpy(memory_image, memory_gene)
    epi = _make_epilogue(B, KP1, D, N)
    out_i, out_g, new_i, new_g = epi(dots_i, dots_g, old_i, old_g, image, gene,
                                     index, cpy_i, cpy_g)
    return (out_i[:, :, None], out_g[:, :, None], new_i, new_g)
